# baseline (speedup 1.0000x reference)
import functools

import jax
import jax.numpy as jnp
from jax import lax
from jax.experimental import pallas as pl
from jax.experimental.pallas import tpu as pltpu
from jax.experimental.pallas import tpu_sc as plsc

CAPACITY = 100000
INPUT_DIM = 512
VALUE_DIM = 256
THRESHOLD = 0.85 + 0.05
EPS = 1e-12
SCALE = 10.0


BLOCK = 4000
NBLK = CAPACITY // BLOCK


def _tc_body(q_ref, k_ref, v_ref, recall_ref, best_ref, acc_ref, m_ref, l_ref):
    i = pl.program_id(0)

    @pl.when(i == 0)
    def _init():
        m_ref[0, 0] = -jnp.inf
        l_ref[0, 0] = 0.0
        acc_ref[...] = jnp.zeros_like(acc_ref)

    q = q_ref[0, :]
    qn = q / jnp.maximum(jnp.sqrt(jnp.sum(q * q)), EPS)

    s = jax.lax.dot_general(
        qn[None, :], k_ref[...],
        dimension_numbers=(((1,), (1,)), ((), ())),
        preferred_element_type=jnp.float32,
    )

    m_prev = m_ref[0, 0]
    m_new = jnp.maximum(m_prev, jnp.max(s))
    c = jnp.exp(SCALE * (m_prev - m_new))
    p = jnp.exp(SCALE * (s - m_new))
    l_ref[0, 0] = l_ref[0, 0] * c + jnp.sum(p)
    pv = jax.lax.dot_general(
        p, v_ref[...],
        dimension_numbers=(((1,), (0,)), ((), ())),
        preferred_element_type=jnp.float32,
    )
    acc_ref[...] = acc_ref[...] * c + pv
    m_ref[0, 0] = m_new

    @pl.when(i == NBLK - 1)
    def _fin():
        best = m_ref[0, 0]
        r = acc_ref[...] / l_ref[0, 0]
        recall_ref[...] = jnp.where(best >= THRESHOLD, r, jnp.zeros_like(r))
        best_ref[...] = jnp.full((1, 1), best, dtype=jnp.float32)


@jax.jit
def _kernel_tc(query_pattern, keys, values):
    q2 = query_pattern.reshape(1, INPUT_DIM)
    recall, best = pl.pallas_call(
        _tc_body,
        grid=(NBLK,),
        in_specs=[
            pl.BlockSpec((1, INPUT_DIM), lambda i: (0, 0)),
            pl.BlockSpec((BLOCK, INPUT_DIM), lambda i: (i, 0)),
            pl.BlockSpec((BLOCK, VALUE_DIM), lambda i: (i, 0)),
        ],
        out_specs=[
            pl.BlockSpec((1, VALUE_DIM), lambda i: (0, 0)),
            pl.BlockSpec((1, 1), lambda i: (0, 0)),
        ],
        out_shape=[
            jax.ShapeDtypeStruct((1, VALUE_DIM), jnp.float32),
            jax.ShapeDtypeStruct((1, 1), jnp.float32),
        ],
        scratch_shapes=[
            pltpu.VMEM((1, VALUE_DIM), jnp.float32),
            pltpu.SMEM((1, 1), jnp.float32),
            pltpu.SMEM((1, 1), jnp.float32),
        ],
        compiler_params=pltpu.CompilerParams(
            dimension_semantics=("arbitrary",),
        ),
    )(q2, keys, values)
    return recall[0], best[0, 0]


NC = 2
NS = 16
NW = NC * NS
ROWS_C = 80
NCH_TOT = CAPACITY // ROWS_C
NCH_BASE = NCH_TOT // NW
NXTRA = NCH_TOT - NCH_BASE * NW
NKC = INPUT_DIM // 16
NVC = VALUE_DIM // 16
NSB = ROWS_C // 16
NEG = -1e30


def _sc_body(q_hbm, keys_hbm, values_hbm, outs_hbm, outv_hbm,
             qbuf, kbuf, vbuf, wbuf, vacc, lbuf, cbuf, m_s):
    wid = lax.axis_index("s") * NC + lax.axis_index("c")

    pltpu.sync_copy(q_hbm, qbuf)

    def _qsq(c, acc):
        x = qbuf[pl.ds(c * 16, 16)]
        return acc + x * x
    nsq = jnp.maximum(
        jnp.sum(lax.fori_loop(0, NKC, _qsq, jnp.zeros((16,), jnp.float32))),
        1e-30,
    )
    x = jnp.full((16,), nsq, jnp.float32)
    yi = jnp.full((16,), 0x5F3759DF, jnp.int32) - (plsc.bitcast(x, jnp.int32) >> 1)
    y = plsc.bitcast(yi, jnp.float32)
    for _ in range(4):
        y = y * (1.5 - 0.5 * x * y * y)
    rinv = y

    def _qn(c, carry):
        qbuf[pl.ds(c * 16, 16)] = qbuf[pl.ds(c * 16, 16)] * rinv
        return carry
    lax.fori_loop(0, NKC, _qn, 0)

    zero16 = jnp.zeros((16,), jnp.float32)
    for j in range(NVC):
        vacc[pl.ds(j * 16, 16)] = zero16
    lbuf[pl.ds(0, 16)] = zero16
    m_s[0] = jnp.float32(NEG)

    qs = [qbuf[pl.ds(c * 16, 16)] for c in range(NKC)]
    il = lax.iota(jnp.int32, 16)
    negv = jnp.full((16,), NEG, jnp.float32)

    def _process_chunk(c):
        row0 = c * ROWS_C
        pltpu.sync_copy(keys_hbm.at[pl.ds(row0, ROWS_C), :], kbuf)
        pltpu.sync_copy(values_hbm.at[pl.ds(row0, ROWS_C), :], vbuf)

        def _row_dot(r, carry2):
            zvec, m_run = carry2
            acc = qs[0] * kbuf[r, pl.ds(0, 16)]
            for c2 in range(1, NKC):
                acc = acc + qs[c2] * kbuf[r, pl.ds(c2 * 16, 16)]
            s = jnp.sum(acc)
            m_run = jnp.maximum(m_run, s)
            grp = (r // 16) * 16
            lane = r - grp
            zvec = jnp.where(il == lane,
                             jnp.full((16,), s * SCALE, jnp.float32), zvec)
            flush = lane == 15

            @pl.when(flush)
            def _():
                wbuf[pl.ds(grp, 16)] = jnp.exp(zvec)

            zvec = jnp.where(flush, negv, zvec)
            return (zvec, m_run)

        _, m_new = lax.fori_loop(0, ROWS_C, _row_dot, (negv, m_s[0]))
        m_s[0] = m_new

        def _wsum(k, acc):
            return acc + wbuf[pl.ds(k * 16, 16)]
        lbuf[pl.ds(0, 16)] = lax.fori_loop(0, NSB, _wsum, lbuf[pl.ds(0, 16)])

        def _vgrp(g, accs):
            wg = wbuf[pl.ds(g * 16, 16)]
            grp = g * 16
            for lane in range(16):
                wr = wg[lane]
                accs = tuple(accs[j] + wr * vbuf[grp + lane, pl.ds(j * 16, 16)]
                             for j in range(NVC))
            return accs
        accs = lax.fori_loop(0, NSB, _vgrp,
                             tuple(vacc[pl.ds(j * 16, 16)]
                                   for j in range(NVC)))
        for j in range(NVC):
            vacc[pl.ds(j * 16, 16)] = accs[j]

    def _iter(i, carry):
        _process_chunk(wid + NW * i)
        return carry
    lax.fori_loop(0, NCH_BASE, _iter, 0)

    @pl.when(wid < NXTRA)
    def _extra():
        _process_chunk(NCH_BASE * NW + wid)

    lsum = jnp.sum(lbuf[pl.ds(0, 16)])
    m_fin = m_s[0]
    sv = jnp.where(il == 0, jnp.full((16,), m_fin, jnp.float32),
                   jnp.where(il == 1, jnp.full((16,), lsum, jnp.float32),
                             jnp.zeros((16,), jnp.float32)))
    cbuf[pl.ds(0, 16)] = sv
    pltpu.sync_copy(cbuf, outs_hbm.at[pl.ds(wid * 16, 16)])
    pltpu.sync_copy(vacc, outv_hbm.at[pl.ds(wid * VALUE_DIM, VALUE_DIM)])


_sc_partials = pl.kernel(
    _sc_body,
    out_type=[
        jax.ShapeDtypeStruct((NW * 16,), jnp.float32),
        jax.ShapeDtypeStruct((NW * VALUE_DIM,), jnp.float32),
    ],
    mesh=plsc.VectorSubcoreMesh(
        core_axis_name="c", subcore_axis_name="s",
        num_cores=NC, num_subcores=NS),
    compiler_params=pltpu.CompilerParams(needs_layout_passes=False),
    scratch_types=[
        pltpu.VMEM((INPUT_DIM,), jnp.float32),
        pltpu.VMEM((ROWS_C, INPUT_DIM), jnp.float32),
        pltpu.VMEM((ROWS_C, VALUE_DIM), jnp.float32),
        pltpu.VMEM((NSB * 16,), jnp.float32),
        pltpu.VMEM((VALUE_DIM,), jnp.float32),
        pltpu.VMEM((16,), jnp.float32),
        pltpu.VMEM((16,), jnp.float32),
        pltpu.SMEM((1,), jnp.float32),
    ],
)


def _combine_body(s_ref, v_ref, recall_ref, best_ref):
    s = s_ref[...]
    m = s[:, 0:1]
    l = s[:, 1:2]
    best = jnp.max(m)
    l_g = jnp.sum(l)
    numer = jnp.sum(v_ref[...], axis=0, keepdims=True)
    r = numer / l_g
    recall_ref[...] = jnp.where(best >= THRESHOLD, r, jnp.zeros_like(r))
    best_ref[...] = jnp.full((1, 1), best, dtype=jnp.float32)


def _combine(parts_s, parts_v):
    return pl.pallas_call(
        _combine_body,
        out_shape=[
            jax.ShapeDtypeStruct((1, VALUE_DIM), jnp.float32),
            jax.ShapeDtypeStruct((1, 1), jnp.float32),
        ],
    )(parts_s, parts_v)


@jax.jit
def _kernel_sc(query_pattern, keys, values):
    parts_s, parts_v = _sc_partials(query_pattern, keys, values)
    recall, best = _combine(parts_s.reshape(NW, 16),
                            parts_v.reshape(NW, VALUE_DIM))
    return recall[0], best[0, 0]


kernel = _kernel_sc


# speedup vs baseline: 3.7476x; 3.7476x over previous
import functools

import jax
import jax.numpy as jnp
from jax import lax
from jax.experimental import pallas as pl
from jax.experimental.pallas import tpu as pltpu
from jax.experimental.pallas import tpu_sc as plsc

CAPACITY = 100000
INPUT_DIM = 512
VALUE_DIM = 256
THRESHOLD = 0.85 + 0.05
EPS = 1e-12
SCALE = 10.0

NC = 2
NS = 16
NW = NC * NS
ROWS_C = 80
BLOCK = 4000
SC_CHUNKS = 200
SC_ROWS = SC_CHUNKS * ROWS_C
TC_ROWS = CAPACITY - SC_ROWS
TC_NBLK = TC_ROWS // BLOCK
TC_OFF = SC_ROWS // BLOCK

NKC = INPUT_DIM // 16
NVC = VALUE_DIM // 16
NSB = ROWS_C // 16
NEG = -1e30

assert SC_ROWS % BLOCK == 0 and TC_ROWS % BLOCK == 0


def _make_sc_partials(nch):
    nch_base = nch // NW
    nxtra = nch - nch_base * NW

    def _sc_body(q_hbm, keys_hbm, values_hbm, outs_hbm, outv_hbm,
                 qbuf, kbuf, vbuf, wbuf, vacc, lbuf, cbuf, m_s):
        wid = lax.axis_index("s") * NC + lax.axis_index("c")

        pltpu.sync_copy(q_hbm, qbuf)

        def _qsq(c, acc):
            x = qbuf[pl.ds(c * 16, 16)]
            return acc + x * x
        nsq = jnp.maximum(
            jnp.sum(lax.fori_loop(0, NKC, _qsq,
                                  jnp.zeros((16,), jnp.float32))),
            1e-30,
        )
        x = jnp.full((16,), nsq, jnp.float32)
        yi = (jnp.full((16,), 0x5F3759DF, jnp.int32)
              - (plsc.bitcast(x, jnp.int32) >> 1))
        y = plsc.bitcast(yi, jnp.float32)
        for _ in range(4):
            y = y * (1.5 - 0.5 * x * y * y)
        rinv = y

        def _qn(c, carry):
            qbuf[pl.ds(c * 16, 16)] = qbuf[pl.ds(c * 16, 16)] * rinv
            return carry
        lax.fori_loop(0, NKC, _qn, 0)

        zero16 = jnp.zeros((16,), jnp.float32)
        for j in range(NVC):
            vacc[pl.ds(j * 16, 16)] = zero16
        lbuf[pl.ds(0, 16)] = zero16
        m_s[0] = jnp.float32(NEG)

        qs = [qbuf[pl.ds(c * 16, 16)] for c in range(NKC)]
        il = lax.iota(jnp.int32, 16)
        negv = jnp.full((16,), NEG, jnp.float32)

        def _process_chunk(c):
            row0 = c * ROWS_C
            pltpu.sync_copy(keys_hbm.at[pl.ds(row0, ROWS_C), :], kbuf)
            pltpu.sync_copy(values_hbm.at[pl.ds(row0, ROWS_C), :], vbuf)

            def _row_dot(r, carry2):
                zvec, m_run = carry2
                acc = qs[0] * kbuf[r, pl.ds(0, 16)]
                for c2 in range(1, NKC):
                    acc = acc + qs[c2] * kbuf[r, pl.ds(c2 * 16, 16)]
                s = jnp.sum(acc)
                m_run = jnp.maximum(m_run, s)
                grp = (r // 16) * 16
                lane = r - grp
                zvec = jnp.where(il == lane,
                                 jnp.full((16,), s * SCALE, jnp.float32),
                                 zvec)
                flush = lane == 15

                @pl.when(flush)
                def _():
                    wbuf[pl.ds(grp, 16)] = jnp.exp(zvec)

                zvec = jnp.where(flush, negv, zvec)
                return (zvec, m_run)

            _, m_new = lax.fori_loop(0, ROWS_C, _row_dot, (negv, m_s[0]))
            m_s[0] = m_new

            def _wsum(k, acc):
                return acc + wbuf[pl.ds(k * 16, 16)]
            lbuf[pl.ds(0, 16)] = lax.fori_loop(0, NSB, _wsum,
                                               lbuf[pl.ds(0, 16)])

            def _vgrp(g, accs):
                wg = wbuf[pl.ds(g * 16, 16)]
                grp = g * 16
                for lane in range(16):
                    wr = wg[lane]
                    accs = tuple(
                        accs[j] + wr * vbuf[grp + lane, pl.ds(j * 16, 16)]
                        for j in range(NVC))
                return accs
            accs = lax.fori_loop(0, NSB, _vgrp,
                                 tuple(vacc[pl.ds(j * 16, 16)]
                                       for j in range(NVC)))
            for j in range(NVC):
                vacc[pl.ds(j * 16, 16)] = accs[j]

        def _iter(i, carry):
            _process_chunk(wid + NW * i)
            return carry
        lax.fori_loop(0, nch_base, _iter, 0)

        if nxtra:
            @pl.when(wid < nxtra)
            def _extra():
                _process_chunk(nch_base * NW + wid)

        lsum = jnp.sum(lbuf[pl.ds(0, 16)])
        m_fin = m_s[0]
        sv = jnp.where(il == 0, jnp.full((16,), m_fin, jnp.float32),
                       jnp.where(il == 1, jnp.full((16,), lsum, jnp.float32),
                                 jnp.zeros((16,), jnp.float32)))
        cbuf[pl.ds(0, 16)] = sv
        pltpu.sync_copy(cbuf, outs_hbm.at[pl.ds(wid * 16, 16)])
        pltpu.sync_copy(vacc, outv_hbm.at[pl.ds(wid * VALUE_DIM, VALUE_DIM)])

    return pl.kernel(
        _sc_body,
        out_type=[
            jax.ShapeDtypeStruct((NW * 16,), jnp.float32),
            jax.ShapeDtypeStruct((NW * VALUE_DIM,), jnp.float32),
        ],
        mesh=plsc.VectorSubcoreMesh(
            core_axis_name="c", subcore_axis_name="s",
            num_cores=NC, num_subcores=NS),
        compiler_params=pltpu.CompilerParams(needs_layout_passes=False),
        scratch_types=[
            pltpu.VMEM((INPUT_DIM,), jnp.float32),
            pltpu.VMEM((ROWS_C, INPUT_DIM), jnp.float32),
            pltpu.VMEM((ROWS_C, VALUE_DIM), jnp.float32),
            pltpu.VMEM((NSB * 16,), jnp.float32),
            pltpu.VMEM((VALUE_DIM,), jnp.float32),
            pltpu.VMEM((16,), jnp.float32),
            pltpu.VMEM((16,), jnp.float32),
            pltpu.SMEM((1,), jnp.float32),
        ],
    )


_sc_partials = _make_sc_partials(SC_CHUNKS)


def _tcp_body(q_ref, k_ref, v_ref, acc_out, m_out, l_out,
              acc_ref, m_ref, l_ref):
    i = pl.program_id(0)

    @pl.when(i == 0)
    def _init():
        m_ref[0, 0] = NEG
        l_ref[0, 0] = 0.0
        acc_ref[...] = jnp.zeros_like(acc_ref)

    q = q_ref[0, :]
    qn = q / jnp.maximum(jnp.sqrt(jnp.sum(q * q)), EPS)

    s = jax.lax.dot_general(
        qn[None, :], k_ref[...],
        dimension_numbers=(((1,), (1,)), ((), ())),
        preferred_element_type=jnp.float32,
    )
    m_ref[0, 0] = jnp.maximum(m_ref[0, 0], jnp.max(s))
    p = jnp.exp(SCALE * s)
    l_ref[0, 0] = l_ref[0, 0] + jnp.sum(p)
    pv = jax.lax.dot_general(
        p, v_ref[...],
        dimension_numbers=(((1,), (0,)), ((), ())),
        preferred_element_type=jnp.float32,
    )
    acc_ref[...] = acc_ref[...] + pv

    @pl.when(i == TC_NBLK - 1)
    def _fin():
        acc_out[...] = acc_ref[...]
        m_out[...] = jnp.full((1, 1), m_ref[0, 0], dtype=jnp.float32)
        l_out[...] = jnp.full((1, 1), l_ref[0, 0], dtype=jnp.float32)


def _tc_partials(q2, keys, values):
    return pl.pallas_call(
        _tcp_body,
        grid=(TC_NBLK,),
        in_specs=[
            pl.BlockSpec((1, INPUT_DIM), lambda i: (0, 0)),
            pl.BlockSpec((BLOCK, INPUT_DIM), lambda i: (i + TC_OFF, 0)),
            pl.BlockSpec((BLOCK, VALUE_DIM), lambda i: (i + TC_OFF, 0)),
        ],
        out_specs=[
            pl.BlockSpec((1, VALUE_DIM), lambda i: (0, 0)),
            pl.BlockSpec((1, 1), lambda i: (0, 0)),
            pl.BlockSpec((1, 1), lambda i: (0, 0)),
        ],
        out_shape=[
            jax.ShapeDtypeStruct((1, VALUE_DIM), jnp.float32),
            jax.ShapeDtypeStruct((1, 1), jnp.float32),
            jax.ShapeDtypeStruct((1, 1), jnp.float32),
        ],
        scratch_shapes=[
            pltpu.VMEM((1, VALUE_DIM), jnp.float32),
            pltpu.SMEM((1, 1), jnp.float32),
            pltpu.SMEM((1, 1), jnp.float32),
        ],
        compiler_params=pltpu.CompilerParams(
            dimension_semantics=("arbitrary",),
        ),
    )(q2, keys, values)


def _combine_body(s_ref, v_ref, tacc_ref, tm_ref, tl_ref,
                  recall_ref, best_ref):
    s = s_ref[...]
    best = jnp.maximum(jnp.max(s[:, 0:1]), tm_ref[0, 0])
    l_g = jnp.sum(s[:, 1:2]) + tl_ref[0, 0]
    numer = jnp.sum(v_ref[...], axis=0, keepdims=True) + tacc_ref[...]
    r = numer / l_g
    recall_ref[...] = jnp.where(best >= THRESHOLD, r, jnp.zeros_like(r))
    best_ref[...] = jnp.full((1, 1), best, dtype=jnp.float32)


def _combine(parts_s, parts_v, tacc, tm, tl):
    return pl.pallas_call(
        _combine_body,
        out_shape=[
            jax.ShapeDtypeStruct((1, VALUE_DIM), jnp.float32),
            jax.ShapeDtypeStruct((1, 1), jnp.float32),
        ],
    )(parts_s, parts_v, tacc, tm, tl)


@jax.jit
def kernel(query_pattern, keys, values):
    q2 = query_pattern.reshape(1, INPUT_DIM)
    parts_s, parts_v = _sc_partials(query_pattern, keys, values)
    tacc, tm, tl = _tc_partials(q2, keys, values)
    recall, best = _combine(parts_s.reshape(NW, 16),
                            parts_v.reshape(NW, VALUE_DIM),
                            tacc, tm, tl)
    return recall[0], best[0, 0]


# speedup vs baseline: 3.9465x; 1.0531x over previous
import functools

import jax
import jax.numpy as jnp
from jax import lax
from jax.experimental import pallas as pl
from jax.experimental.pallas import tpu as pltpu
from jax.experimental.pallas import tpu_sc as plsc

CAPACITY = 100000
INPUT_DIM = 512
VALUE_DIM = 256
THRESHOLD = 0.85 + 0.05
EPS = 1e-12
SCALE = 10.0

NC = 2
NS = 16
NW = NC * NS
ROWS_C = 80
BLOCK = 4000
SC_CHUNKS = 50
SC_ROWS = SC_CHUNKS * ROWS_C
TC_ROWS = CAPACITY - SC_ROWS
TC_NBLK = TC_ROWS // BLOCK
TC_OFF = SC_ROWS // BLOCK

NKC = INPUT_DIM // 16
NVC = VALUE_DIM // 16
NSB = ROWS_C // 16
NEG = -1e30

assert SC_ROWS % BLOCK == 0 and TC_ROWS % BLOCK == 0


def _make_sc_partials(nch):
    nch_base = nch // NW
    nxtra = nch - nch_base * NW

    def _sc_body(q_hbm, keys_hbm, values_hbm, outs_hbm, outv_hbm,
                 qbuf, kbuf, vbuf, wbuf, vacc, lbuf, cbuf, m_s):
        wid = lax.axis_index("s") * NC + lax.axis_index("c")

        pltpu.sync_copy(q_hbm, qbuf)

        def _qsq(c, acc):
            x = qbuf[pl.ds(c * 16, 16)]
            return acc + x * x
        nsq = jnp.maximum(
            jnp.sum(lax.fori_loop(0, NKC, _qsq,
                                  jnp.zeros((16,), jnp.float32))),
            1e-30,
        )
        x = jnp.full((16,), nsq, jnp.float32)
        yi = (jnp.full((16,), 0x5F3759DF, jnp.int32)
              - (plsc.bitcast(x, jnp.int32) >> 1))
        y = plsc.bitcast(yi, jnp.float32)
        for _ in range(4):
            y = y * (1.5 - 0.5 * x * y * y)
        rinv = y

        def _qn(c, carry):
            qbuf[pl.ds(c * 16, 16)] = qbuf[pl.ds(c * 16, 16)] * rinv
            return carry
        lax.fori_loop(0, NKC, _qn, 0)

        zero16 = jnp.zeros((16,), jnp.float32)
        for j in range(NVC):
            vacc[pl.ds(j * 16, 16)] = zero16
        lbuf[pl.ds(0, 16)] = zero16
        m_s[0] = jnp.float32(NEG)

        qs = [qbuf[pl.ds(c * 16, 16)] for c in range(NKC)]
        il = lax.iota(jnp.int32, 16)
        negv = jnp.full((16,), NEG, jnp.float32)

        def _process_chunk(c):
            row0 = c * ROWS_C
            pltpu.sync_copy(keys_hbm.at[pl.ds(row0, ROWS_C), :], kbuf)
            pltpu.sync_copy(values_hbm.at[pl.ds(row0, ROWS_C), :], vbuf)

            def _row_dot(r, carry2):
                zvec, m_run = carry2
                acc = qs[0] * kbuf[r, pl.ds(0, 16)]
                for c2 in range(1, NKC):
                    acc = acc + qs[c2] * kbuf[r, pl.ds(c2 * 16, 16)]
                s = jnp.sum(acc)
                m_run = jnp.maximum(m_run, s)
                grp = (r // 16) * 16
                lane = r - grp
                zvec = jnp.where(il == lane,
                                 jnp.full((16,), s * SCALE, jnp.float32),
                                 zvec)
                flush = lane == 15

                @pl.when(flush)
                def _():
                    wbuf[pl.ds(grp, 16)] = jnp.exp(zvec)

                zvec = jnp.where(flush, negv, zvec)
                return (zvec, m_run)

            _, m_new = lax.fori_loop(0, ROWS_C, _row_dot, (negv, m_s[0]))
            m_s[0] = m_new

            def _wsum(k, acc):
                return acc + wbuf[pl.ds(k * 16, 16)]
            lbuf[pl.ds(0, 16)] = lax.fori_loop(0, NSB, _wsum,
                                               lbuf[pl.ds(0, 16)])

            def _vgrp(g, accs):
                wg = wbuf[pl.ds(g * 16, 16)]
                grp = g * 16
                for lane in range(16):
                    wr = wg[lane]
                    accs = tuple(
                        accs[j] + wr * vbuf[grp + lane, pl.ds(j * 16, 16)]
                        for j in range(NVC))
                return accs
            accs = lax.fori_loop(0, NSB, _vgrp,
                                 tuple(vacc[pl.ds(j * 16, 16)]
                                       for j in range(NVC)))
            for j in range(NVC):
                vacc[pl.ds(j * 16, 16)] = accs[j]

        def _iter(i, carry):
            _process_chunk(wid + NW * i)
            return carry
        lax.fori_loop(0, nch_base, _iter, 0)

        if nxtra:
            @pl.when(wid < nxtra)
            def _extra():
                _process_chunk(nch_base * NW + wid)

        lsum = jnp.sum(lbuf[pl.ds(0, 16)])
        m_fin = m_s[0]
        sv = jnp.where(il == 0, jnp.full((16,), m_fin, jnp.float32),
                       jnp.where(il == 1, jnp.full((16,), lsum, jnp.float32),
                                 jnp.zeros((16,), jnp.float32)))
        cbuf[pl.ds(0, 16)] = sv
        pltpu.sync_copy(cbuf, outs_hbm.at[pl.ds(wid * 16, 16)])
        pltpu.sync_copy(vacc, outv_hbm.at[pl.ds(wid * VALUE_DIM, VALUE_DIM)])

    return pl.kernel(
        _sc_body,
        out_type=[
            jax.ShapeDtypeStruct((NW * 16,), jnp.float32),
            jax.ShapeDtypeStruct((NW * VALUE_DIM,), jnp.float32),
        ],
        mesh=plsc.VectorSubcoreMesh(
            core_axis_name="c", subcore_axis_name="s",
            num_cores=NC, num_subcores=NS),
        compiler_params=pltpu.CompilerParams(needs_layout_passes=False),
        scratch_types=[
            pltpu.VMEM((INPUT_DIM,), jnp.float32),
            pltpu.VMEM((ROWS_C, INPUT_DIM), jnp.float32),
            pltpu.VMEM((ROWS_C, VALUE_DIM), jnp.float32),
            pltpu.VMEM((NSB * 16,), jnp.float32),
            pltpu.VMEM((VALUE_DIM,), jnp.float32),
            pltpu.VMEM((16,), jnp.float32),
            pltpu.VMEM((16,), jnp.float32),
            pltpu.SMEM((1,), jnp.float32),
        ],
    )


_sc_partials = _make_sc_partials(SC_CHUNKS)


def _tcp_body(q_ref, k_ref, v_ref, acc_out, m_out, l_out,
              acc_ref, m_ref, l_ref):
    i = pl.program_id(0)

    @pl.when(i == 0)
    def _init():
        m_ref[0, 0] = NEG
        l_ref[0, 0] = 0.0
        acc_ref[...] = jnp.zeros_like(acc_ref)

    q = q_ref[0, :]
    qn = q / jnp.maximum(jnp.sqrt(jnp.sum(q * q)), EPS)

    s = jax.lax.dot_general(
        qn[None, :], k_ref[...],
        dimension_numbers=(((1,), (1,)), ((), ())),
        preferred_element_type=jnp.float32,
    )
    m_ref[0, 0] = jnp.maximum(m_ref[0, 0], jnp.max(s))
    p = jnp.exp(SCALE * s)
    l_ref[0, 0] = l_ref[0, 0] + jnp.sum(p)
    pv = jax.lax.dot_general(
        p, v_ref[...],
        dimension_numbers=(((1,), (0,)), ((), ())),
        preferred_element_type=jnp.float32,
    )
    acc_ref[...] = acc_ref[...] + pv

    @pl.when(i == TC_NBLK - 1)
    def _fin():
        acc_out[...] = acc_ref[...]
        m_out[...] = jnp.full((1, 1), m_ref[0, 0], dtype=jnp.float32)
        l_out[...] = jnp.full((1, 1), l_ref[0, 0], dtype=jnp.float32)


def _tc_partials(q2, keys, values):
    return pl.pallas_call(
        _tcp_body,
        grid=(TC_NBLK,),
        in_specs=[
            pl.BlockSpec((1, INPUT_DIM), lambda i: (0, 0)),
            pl.BlockSpec((BLOCK, INPUT_DIM), lambda i: (i + TC_OFF, 0)),
            pl.BlockSpec((BLOCK, VALUE_DIM), lambda i: (i + TC_OFF, 0)),
        ],
        out_specs=[
            pl.BlockSpec((1, VALUE_DIM), lambda i: (0, 0)),
            pl.BlockSpec((1, 1), lambda i: (0, 0)),
            pl.BlockSpec((1, 1), lambda i: (0, 0)),
        ],
        out_shape=[
            jax.ShapeDtypeStruct((1, VALUE_DIM), jnp.float32),
            jax.ShapeDtypeStruct((1, 1), jnp.float32),
            jax.ShapeDtypeStruct((1, 1), jnp.float32),
        ],
        scratch_shapes=[
            pltpu.VMEM((1, VALUE_DIM), jnp.float32),
            pltpu.SMEM((1, 1), jnp.float32),
            pltpu.SMEM((1, 1), jnp.float32),
        ],
        compiler_params=pltpu.CompilerParams(
            dimension_semantics=("arbitrary",),
        ),
    )(q2, keys, values)


def _combine_body(s_ref, v_ref, tacc_ref, tm_ref, tl_ref,
                  recall_ref, best_ref):
    s = s_ref[...]
    best = jnp.maximum(jnp.max(s[:, 0:1]), tm_ref[0, 0])
    l_g = jnp.sum(s[:, 1:2]) + tl_ref[0, 0]
    numer = jnp.sum(v_ref[...], axis=0, keepdims=True) + tacc_ref[...]
    r = numer / l_g
    recall_ref[...] = jnp.where(best >= THRESHOLD, r, jnp.zeros_like(r))
    best_ref[...] = jnp.full((1, 1), best, dtype=jnp.float32)


def _combine(parts_s, parts_v, tacc, tm, tl):
    return pl.pallas_call(
        _combine_body,
        out_shape=[
            jax.ShapeDtypeStruct((1, VALUE_DIM), jnp.float32),
            jax.ShapeDtypeStruct((1, 1), jnp.float32),
        ],
    )(parts_s, parts_v, tacc, tm, tl)


@jax.jit
def kernel(query_pattern, keys, values):
    q2 = query_pattern.reshape(1, INPUT_DIM)
    parts_s, parts_v = _sc_partials(query_pattern, keys, values)
    tacc, tm, tl = _tc_partials(q2, keys, values)
    recall, best = _combine(parts_s.reshape(NW, 16),
                            parts_v.reshape(NW, VALUE_DIM),
                            tacc, tm, tl)
    return recall[0], best[0, 0]


# speedup vs baseline: 3.9816x; 1.0089x over previous
import functools

import jax
import jax.numpy as jnp
from jax import lax
from jax.experimental import pallas as pl
from jax.experimental.pallas import tpu as pltpu
from jax.experimental.pallas import tpu_sc as plsc

CAPACITY = 100000
INPUT_DIM = 512
VALUE_DIM = 256
THRESHOLD = 0.85 + 0.05
EPS = 1e-12
SCALE = 10.0

NC = 2
NS = 16
NW = NC * NS
ROWS_C = 80
BLOCK = 4000
SC_CHUNKS = 300
SC_ROWS = SC_CHUNKS * ROWS_C
TC_ROWS = CAPACITY - SC_ROWS
TC_NBLK = TC_ROWS // BLOCK
TC_OFF = SC_ROWS // BLOCK

NKC = INPUT_DIM // 16
NVC = VALUE_DIM // 16
NSB = ROWS_C // 16
NEG = -1e30

assert SC_ROWS % BLOCK == 0 and TC_ROWS % BLOCK == 0


def _make_sc_partials(nch):
    nch_base = nch // NW
    nxtra = nch - nch_base * NW

    def _sc_body(q_hbm, keys_hbm, values_hbm, outs_hbm, outv_hbm,
                 qbuf, kbufA, kbufB, vbufA, vbufB, wbuf, vacc, lbuf, cbuf,
                 m_s, ksemA, ksemB, vsemA, vsemB):
        wid = lax.axis_index("s") * NC + lax.axis_index("c")

        pltpu.sync_copy(q_hbm, qbuf)

        def _qsq(c, acc):
            x = qbuf[pl.ds(c * 16, 16)]
            return acc + x * x
        nsq = jnp.maximum(
            jnp.sum(lax.fori_loop(0, NKC, _qsq,
                                  jnp.zeros((16,), jnp.float32))),
            1e-30,
        )
        x = jnp.full((16,), nsq, jnp.float32)
        yi = (jnp.full((16,), 0x5F3759DF, jnp.int32)
              - (plsc.bitcast(x, jnp.int32) >> 1))
        y = plsc.bitcast(yi, jnp.float32)
        for _ in range(4):
            y = y * (1.5 - 0.5 * x * y * y)
        rinv = y

        def _qn(c, carry):
            qbuf[pl.ds(c * 16, 16)] = qbuf[pl.ds(c * 16, 16)] * rinv
            return carry
        lax.fori_loop(0, NKC, _qn, 0)

        zero16 = jnp.zeros((16,), jnp.float32)
        for j in range(NVC):
            vacc[pl.ds(j * 16, 16)] = zero16
        lbuf[pl.ds(0, 16)] = zero16
        m_s[0] = jnp.float32(NEG)

        qs = [qbuf[pl.ds(c * 16, 16)] for c in range(NKC)]
        il = lax.iota(jnp.int32, 16)
        negv = jnp.full((16,), NEG, jnp.float32)

        n_w = nch_base + jnp.where(wid < nxtra, 1, 0)

        def _chunk_id(i):
            return jnp.where(i < nch_base, wid + NW * i, nch_base * NW + wid)

        def _compute_chunk(kbuf, vbuf):
            def _row_dot(r, carry2):
                zvec, m_run = carry2
                acc = qs[0] * kbuf[r, pl.ds(0, 16)]
                for c2 in range(1, NKC):
                    acc = acc + qs[c2] * kbuf[r, pl.ds(c2 * 16, 16)]
                s = jnp.sum(acc)
                m_run = jnp.maximum(m_run, s)
                grp = (r // 16) * 16
                lane = r - grp
                zvec = jnp.where(il == lane,
                                 jnp.full((16,), s * SCALE, jnp.float32),
                                 zvec)
                flush = lane == 15

                @pl.when(flush)
                def _():
                    wbuf[pl.ds(grp, 16)] = jnp.exp(zvec)

                zvec = jnp.where(flush, negv, zvec)
                return (zvec, m_run)

            _, m_new = lax.fori_loop(0, ROWS_C, _row_dot, (negv, m_s[0]))
            m_s[0] = m_new

            def _wsum(k, acc):
                return acc + wbuf[pl.ds(k * 16, 16)]
            lbuf[pl.ds(0, 16)] = lax.fori_loop(0, NSB, _wsum,
                                               lbuf[pl.ds(0, 16)])

            def _vgrp(g, accs):
                wg = wbuf[pl.ds(g * 16, 16)]
                grp = g * 16
                for lane in range(16):
                    wr = wg[lane]
                    accs = tuple(
                        accs[j] + wr * vbuf[grp + lane, pl.ds(j * 16, 16)]
                        for j in range(NVC))
                return accs
            accs = lax.fori_loop(0, NSB, _vgrp,
                                 tuple(vacc[pl.ds(j * 16, 16)]
                                       for j in range(NVC)))
            for j in range(NVC):
                vacc[pl.ds(j * 16, 16)] = accs[j]

        def _start(c, kbuf, vbuf, ksem, vsem):
            row0 = c * ROWS_C
            pltpu.async_copy(keys_hbm.at[pl.ds(row0, ROWS_C), :], kbuf, ksem)
            pltpu.async_copy(values_hbm.at[pl.ds(row0, ROWS_C), :], vbuf,
                             vsem)

        def _phase(i, kbuf, vbuf, ksem, vsem, kbuf2, vbuf2, ksem2, vsem2):
            pltpu.make_async_copy(
                keys_hbm.at[pl.ds(0, ROWS_C), :], kbuf, ksem).wait()
            pltpu.make_async_copy(
                values_hbm.at[pl.ds(0, ROWS_C), :], vbuf, vsem).wait()

            @pl.when(i + 1 < n_w)
            def _pref():
                _start(_chunk_id(i + 1), kbuf2, vbuf2, ksem2, vsem2)

            _compute_chunk(kbuf, vbuf)

        _start(_chunk_id(0), kbufA, vbufA, ksemA, vsemA)

        def _iter(i, carry):
            @pl.when(i % 2 == 0)
            def _even():
                _phase(i, kbufA, vbufA, ksemA, vsemA,
                       kbufB, vbufB, ksemB, vsemB)

            @pl.when(i % 2 == 1)
            def _odd():
                _phase(i, kbufB, vbufB, ksemB, vsemB,
                       kbufA, vbufA, ksemA, vsemA)
            return carry
        lax.fori_loop(0, n_w, _iter, 0)

        lsum = jnp.sum(lbuf[pl.ds(0, 16)])
        m_fin = m_s[0]
        sv = jnp.where(il == 0, jnp.full((16,), m_fin, jnp.float32),
                       jnp.where(il == 1, jnp.full((16,), lsum, jnp.float32),
                                 jnp.zeros((16,), jnp.float32)))
        cbuf[pl.ds(0, 16)] = sv
        pltpu.sync_copy(cbuf, outs_hbm.at[pl.ds(wid * 16, 16)])
        pltpu.sync_copy(vacc, outv_hbm.at[pl.ds(wid * VALUE_DIM, VALUE_DIM)])

    return pl.kernel(
        _sc_body,
        out_type=[
            jax.ShapeDtypeStruct((NW * 16,), jnp.float32),
            jax.ShapeDtypeStruct((NW * VALUE_DIM,), jnp.float32),
        ],
        mesh=plsc.VectorSubcoreMesh(
            core_axis_name="c", subcore_axis_name="s",
            num_cores=NC, num_subcores=NS),
        compiler_params=pltpu.CompilerParams(needs_layout_passes=False),
        scratch_types=[
            pltpu.VMEM((INPUT_DIM,), jnp.float32),
            pltpu.VMEM((ROWS_C, INPUT_DIM), jnp.float32),
            pltpu.VMEM((ROWS_C, INPUT_DIM), jnp.float32),
            pltpu.VMEM((ROWS_C, VALUE_DIM), jnp.float32),
            pltpu.VMEM((ROWS_C, VALUE_DIM), jnp.float32),
            pltpu.VMEM((NSB * 16,), jnp.float32),
            pltpu.VMEM((VALUE_DIM,), jnp.float32),
            pltpu.VMEM((16,), jnp.float32),
            pltpu.VMEM((16,), jnp.float32),
            pltpu.SMEM((1,), jnp.float32),
            pltpu.SemaphoreType.DMA,
            pltpu.SemaphoreType.DMA,
            pltpu.SemaphoreType.DMA,
            pltpu.SemaphoreType.DMA,
        ],
    )


_sc_partials = _make_sc_partials(SC_CHUNKS)


def _tcp_body(q_ref, k_ref, v_ref, acc_out, m_out, l_out,
              acc_ref, m_ref, l_ref):
    i = pl.program_id(0)

    @pl.when(i == 0)
    def _init():
        m_ref[0, 0] = NEG
        l_ref[0, 0] = 0.0
        acc_ref[...] = jnp.zeros_like(acc_ref)

    q = q_ref[0, :]
    qn = q / jnp.maximum(jnp.sqrt(jnp.sum(q * q)), EPS)

    s = jax.lax.dot_general(
        qn[None, :], k_ref[...],
        dimension_numbers=(((1,), (1,)), ((), ())),
        preferred_element_type=jnp.float32,
    )
    m_ref[0, 0] = jnp.maximum(m_ref[0, 0], jnp.max(s))
    p = jnp.exp(SCALE * s)
    l_ref[0, 0] = l_ref[0, 0] + jnp.sum(p)
    pv = jax.lax.dot_general(
        p, v_ref[...],
        dimension_numbers=(((1,), (0,)), ((), ())),
        preferred_element_type=jnp.float32,
    )
    acc_ref[...] = acc_ref[...] + pv

    @pl.when(i == TC_NBLK - 1)
    def _fin():
        acc_out[...] = acc_ref[...]
        m_out[...] = jnp.full((1, 1), m_ref[0, 0], dtype=jnp.float32)
        l_out[...] = jnp.full((1, 1), l_ref[0, 0], dtype=jnp.float32)


def _tc_partials(q2, keys, values):
    return pl.pallas_call(
        _tcp_body,
        grid=(TC_NBLK,),
        in_specs=[
            pl.BlockSpec((1, INPUT_DIM), lambda i: (0, 0)),
            pl.BlockSpec((BLOCK, INPUT_DIM), lambda i: (i + TC_OFF, 0)),
            pl.BlockSpec((BLOCK, VALUE_DIM), lambda i: (i + TC_OFF, 0)),
        ],
        out_specs=[
            pl.BlockSpec((1, VALUE_DIM), lambda i: (0, 0)),
            pl.BlockSpec((1, 1), lambda i: (0, 0)),
            pl.BlockSpec((1, 1), lambda i: (0, 0)),
        ],
        out_shape=[
            jax.ShapeDtypeStruct((1, VALUE_DIM), jnp.float32),
            jax.ShapeDtypeStruct((1, 1), jnp.float32),
            jax.ShapeDtypeStruct((1, 1), jnp.float32),
        ],
        scratch_shapes=[
            pltpu.VMEM((1, VALUE_DIM), jnp.float32),
            pltpu.SMEM((1, 1), jnp.float32),
            pltpu.SMEM((1, 1), jnp.float32),
        ],
        compiler_params=pltpu.CompilerParams(
            dimension_semantics=("arbitrary",),
        ),
    )(q2, keys, values)


def _combine_body(s_ref, v_ref, tacc_ref, tm_ref, tl_ref,
                  recall_ref, best_ref):
    sv = s_ref[...].reshape(1, NW * 16)
    idx = jax.lax.broadcasted_iota(jnp.int32, (1, NW * 16), 1)
    best = jnp.maximum(jnp.max(jnp.where(idx % 16 == 0, sv, NEG)),
                       tm_ref[0, 0])
    l_g = jnp.sum(jnp.where(idx % 16 == 1, sv, 0.0)) + tl_ref[0, 0]
    acc = tacc_ref[0, :]
    for j in range(NW):
        acc = acc + v_ref[pl.ds(j * VALUE_DIM, VALUE_DIM)]
    r = (acc / l_g)[None, :]
    recall_ref[...] = jnp.where(best >= THRESHOLD, r, jnp.zeros_like(r))
    best_ref[...] = jnp.full((1, 1), best, dtype=jnp.float32)


def _combine(parts_s, parts_v, tacc, tm, tl):
    return pl.pallas_call(
        _combine_body,
        out_shape=[
            jax.ShapeDtypeStruct((1, VALUE_DIM), jnp.float32),
            jax.ShapeDtypeStruct((1, 1), jnp.float32),
        ],
    )(parts_s, parts_v, tacc, tm, tl)


@jax.jit
def kernel(query_pattern, keys, values):
    q2 = query_pattern.reshape(1, INPUT_DIM)
    parts_s, parts_v = _sc_partials(query_pattern, keys, values)
    tacc, tm, tl = _tc_partials(q2, keys, values)
    recall, best = _combine(parts_s, parts_v, tacc, tm, tl)
    return recall[0], best[0, 0]


# speedup vs baseline: 4.5150x; 1.1340x over previous
import functools

import jax
import jax.numpy as jnp
from jax.experimental import pallas as pl
from jax.experimental.pallas import tpu as pltpu

CAPACITY = 100000
INPUT_DIM = 512
VALUE_DIM = 256
THRESHOLD = 0.85 + 0.05
EPS = 1e-12
SCALE = 10.0

BLOCK = 10000
NBLK = CAPACITY // BLOCK


def _body(q_ref, k_ref, v_ref, recall_ref, best_ref, acc_ref, m_ref, l_ref):
    i = pl.program_id(0)

    @pl.when(i == 0)
    def _init():
        m_ref[0, 0] = -jnp.inf
        l_ref[0, 0] = 0.0
        acc_ref[...] = jnp.zeros_like(acc_ref)

    q = q_ref[0, :]
    qn = q / jnp.maximum(jnp.sqrt(jnp.sum(q * q)), EPS)

    s = jax.lax.dot_general(
        qn[None, :], k_ref[...],
        dimension_numbers=(((1,), (1,)), ((), ())),
        preferred_element_type=jnp.float32,
    )

    m_prev = m_ref[0, 0]
    m_new = jnp.maximum(m_prev, jnp.max(s))
    c = jnp.exp(SCALE * (m_prev - m_new))
    p = jnp.exp(SCALE * (s - m_new))
    l_ref[0, 0] = l_ref[0, 0] * c + jnp.sum(p)
    pv = jax.lax.dot_general(
        p, v_ref[...],
        dimension_numbers=(((1,), (0,)), ((), ())),
        preferred_element_type=jnp.float32,
    )
    acc_ref[...] = acc_ref[...] * c + pv
    m_ref[0, 0] = m_new

    @pl.when(i == NBLK - 1)
    def _fin():
        best = m_ref[0, 0]
        r = acc_ref[...] / l_ref[0, 0]
        recall_ref[...] = jnp.where(best >= THRESHOLD, r, jnp.zeros_like(r))
        best_ref[...] = jnp.full((1, 1), best, dtype=jnp.float32)


@jax.jit
def kernel(query_pattern, keys, values):
    q2 = query_pattern.reshape(1, INPUT_DIM)
    recall, best = pl.pallas_call(
        _body,
        grid=(NBLK,),
        in_specs=[
            pl.BlockSpec((1, INPUT_DIM), lambda i: (0, 0)),
            pl.BlockSpec((BLOCK, INPUT_DIM), lambda i: (i, 0)),
            pl.BlockSpec((BLOCK, VALUE_DIM), lambda i: (i, 0)),
        ],
        out_specs=[
            pl.BlockSpec((1, VALUE_DIM), lambda i: (0, 0)),
            pl.BlockSpec((1, 1), lambda i: (0, 0)),
        ],
        out_shape=[
            jax.ShapeDtypeStruct((1, VALUE_DIM), jnp.float32),
            jax.ShapeDtypeStruct((1, 1), jnp.float32),
        ],
        scratch_shapes=[
            pltpu.VMEM((1, VALUE_DIM), jnp.float32),
            pltpu.SMEM((1, 1), jnp.float32),
            pltpu.SMEM((1, 1), jnp.float32),
        ],
        compiler_params=pltpu.CompilerParams(
            dimension_semantics=("arbitrary",),
            vmem_limit_bytes=110 * 1024 * 1024,
        ),
    )(q2, keys, values)
    return recall[0], best[0, 0]


# speedup vs baseline: 4.7824x; 1.0592x over previous
import functools

import jax
import jax.numpy as jnp
from jax.experimental import pallas as pl
from jax.experimental.pallas import tpu as pltpu

CAPACITY = 100000
INPUT_DIM = 512
VALUE_DIM = 256
THRESHOLD = 0.85 + 0.05
EPS = 1e-12
SCALE = 10.0

BLOCK = 4000
NBLK = CAPACITY // BLOCK


def _body(q_ref, k_ref, v_ref, recall_ref, best_ref, acc_ref, m_ref, l_ref,
          qn_ref):
    i = pl.program_id(0)

    @pl.when(i == 0)
    def _init():
        m_ref[0, 0] = -jnp.inf
        l_ref[0, 0] = 0.0
        acc_ref[...] = jnp.zeros_like(acc_ref)
        q = q_ref[...]
        qn_ref[...] = q / jnp.maximum(jnp.sqrt(jnp.sum(q * q)), EPS)

    s = jax.lax.dot_general(
        qn_ref[...], k_ref[...],
        dimension_numbers=(((1,), (1,)), ((), ())),
        preferred_element_type=jnp.float32,
    )

    m_prev = m_ref[0, 0]
    m_new = jnp.maximum(m_prev, jnp.max(s))
    c = jnp.exp(SCALE * (m_prev - m_new))
    p = jnp.exp(SCALE * (s - m_new))
    l_ref[0, 0] = l_ref[0, 0] * c + jnp.sum(p)
    pv = jax.lax.dot_general(
        p, v_ref[...],
        dimension_numbers=(((1,), (0,)), ((), ())),
        preferred_element_type=jnp.float32,
    )
    acc_ref[...] = acc_ref[...] * c + pv
    m_ref[0, 0] = m_new

    @pl.when(i == NBLK - 1)
    def _fin():
        best = m_ref[0, 0]
        r = acc_ref[...] / l_ref[0, 0]
        recall_ref[...] = jnp.where(best >= THRESHOLD, r, jnp.zeros_like(r))
        best_ref[...] = jnp.full((1, 1), best, dtype=jnp.float32)


@jax.jit
def kernel(query_pattern, keys, values):
    q2 = query_pattern.reshape(1, INPUT_DIM)
    recall, best = pl.pallas_call(
        _body,
        grid=(NBLK,),
        in_specs=[
            pl.BlockSpec((1, INPUT_DIM), lambda i: (0, 0)),
            pl.BlockSpec((BLOCK, INPUT_DIM), lambda i: (i, 0)),
            pl.BlockSpec((BLOCK, VALUE_DIM), lambda i: (i, 0)),
        ],
        out_specs=[
            pl.BlockSpec((1, VALUE_DIM), lambda i: (0, 0)),
            pl.BlockSpec((1, 1), lambda i: (0, 0)),
        ],
        out_shape=[
            jax.ShapeDtypeStruct((1, VALUE_DIM), jnp.float32),
            jax.ShapeDtypeStruct((1, 1), jnp.float32),
        ],
        scratch_shapes=[
            pltpu.VMEM((1, VALUE_DIM), jnp.float32),
            pltpu.SMEM((1, 1), jnp.float32),
            pltpu.SMEM((1, 1), jnp.float32),
            pltpu.VMEM((1, INPUT_DIM), jnp.float32),
        ],
        compiler_params=pltpu.CompilerParams(
            dimension_semantics=("arbitrary",),
        ),
    )(q2, keys, values)
    return recall[0], best[0, 0]


# speedup vs baseline: 4.7840x; 1.0003x over previous
import jax
import jax.numpy as jnp
from jax.experimental import pallas as pl
from jax.experimental.pallas import tpu as pltpu

CAPACITY = 100000
INPUT_DIM = 512
VALUE_DIM = 256
THRESHOLD = 0.85 + 0.05
EPS = 1e-12
SCALE = 10.0

BLOCK = 4000
NBLK = CAPACITY // BLOCK


def _body(q_ref, k_ref, v_ref, recall_ref, best_ref, acc_ref, m_ref, l_ref,
          qn_ref):
    i = pl.program_id(0)

    @pl.when(i == 0)
    def _init():
        m_ref[0, 0] = -jnp.inf
        l_ref[0, 0] = 0.0
        acc_ref[...] = jnp.zeros_like(acc_ref)
        q = q_ref[...]
        qn_ref[...] = q / jnp.maximum(jnp.sqrt(jnp.sum(q * q)), EPS)

    s = jax.lax.dot_general(
        qn_ref[...], k_ref[...],
        dimension_numbers=(((1,), (1,)), ((), ())),
        preferred_element_type=jnp.float32,
    )

    m_prev = m_ref[0, 0]
    m_new = jnp.maximum(m_prev, jnp.max(s))
    c = jnp.exp(SCALE * (m_prev - m_new))
    p = jnp.exp(SCALE * (s - m_new))
    l_ref[0, 0] = l_ref[0, 0] * c + jnp.sum(p)
    pv = jax.lax.dot_general(
        p, v_ref[...],
        dimension_numbers=(((1,), (0,)), ((), ())),
        preferred_element_type=jnp.float32,
    )
    acc_ref[...] = acc_ref[...] * c + pv
    m_ref[0, 0] = m_new

    @pl.when(i == NBLK - 1)
    def _fin():
        best = m_ref[0, 0]
        r = acc_ref[...] / l_ref[0, 0]
        recall_ref[...] = jnp.where(best >= THRESHOLD, r, jnp.zeros_like(r))
        best_ref[...] = jnp.full((1, 1), best, dtype=jnp.float32)


@jax.jit
def kernel(query_pattern, keys, values):
    q2 = query_pattern.reshape(1, INPUT_DIM)
    recall, best = pl.pallas_call(
        _body,
        grid=(NBLK,),
        in_specs=[
            pl.BlockSpec((1, INPUT_DIM), lambda i: (0, 0)),
            pl.BlockSpec((BLOCK, INPUT_DIM), lambda i: (i, 0)),
            pl.BlockSpec((BLOCK, VALUE_DIM), lambda i: (i, 0)),
        ],
        out_specs=[
            pl.BlockSpec((1, VALUE_DIM), lambda i: (0, 0)),
            pl.BlockSpec((1, 1), lambda i: (0, 0)),
        ],
        out_shape=[
            jax.ShapeDtypeStruct((1, VALUE_DIM), jnp.float32),
            jax.ShapeDtypeStruct((1, 1), jnp.float32),
        ],
        scratch_shapes=[
            pltpu.VMEM((1, VALUE_DIM), jnp.float32),
            pltpu.SMEM((1, 1), jnp.float32),
            pltpu.SMEM((1, 1), jnp.float32),
            pltpu.VMEM((1, INPUT_DIM), jnp.float32),
        ],
        compiler_params=pltpu.CompilerParams(
            dimension_semantics=("arbitrary",),
        ),
    )(q2, keys, values)
    return recall[0], best[0, 0]
